# baseline (device time: 104483 ns/iter reference)
import jax
import jax.numpy as jnp
from jax import lax
from jax.experimental import pallas as pl
from jax.experimental.pallas import tpu as pltpu

N_DEV = 8
N_HOP = N_DEV - 1
N_SEG = 4
W_SLICES = 8


def kernel(x, w_mat, scale_x, scale_w):
    m_per, k = x.shape
    k2, n_per = w_mat.shape
    assert k == k2
    half = m_per // 2
    seg = half // N_SEG
    ws = k // W_SLICES

    x8 = x.astype(jnp.float8_e5m2)

    def body(x_ref, w_hbm, sx_ref, sw_ref, out_ref,
             w_bf, wbuf, comm_r, comm_l, stage_ref,
             send_r, recv_r, send_l, recv_l, wsems, out_sems):
        my = lax.axis_index("i")
        left = lax.rem(my + N_DEV - 1, N_DEV)
        right = lax.rem(my + 1, N_DEV)

        barrier_sem = pltpu.get_barrier_semaphore()
        for nbr in (left, right):
            pl.semaphore_signal(
                barrier_sem, inc=1,
                device_id=(nbr,), device_id_type=pl.DeviceIdType.MESH,
            )
        pl.semaphore_wait(barrier_sem, 2)

        def mk(d, h, s):
            comm = comm_r if d == 0 else comm_l
            if h == 0:
                src = x_ref.at[pl.ds(d * half + s * seg, seg), :]
            else:
                src = comm.at[h - 1, pl.ds(s * seg, seg), :]
            return pltpu.make_async_remote_copy(
                src_ref=src,
                dst_ref=comm.at[h, pl.ds(s * seg, seg), :],
                send_sem=(send_r if d == 0 else send_l).at[h, s],
                recv_sem=(recv_r if d == 0 else recv_l).at[h, s],
                device_id=(right if d == 0 else left,),
                device_id_type=pl.DeviceIdType.MESH,
            )

        desc = [[[mk(d, h, s) for s in range(N_SEG)] for h in range(N_HOP)]
                for d in range(2)]
        for d in range(2):
            for s in range(N_SEG):
                desc[d][0][s].start()

        wd = [pltpu.make_async_copy(
                  w_hbm.at[pl.ds(i * ws, ws), :], wbuf.at[i % 2],
                  wsems.at[i % 2])
              for i in range(W_SLICES)]
        wd[0].start()
        wd[1].start()
        for i in range(W_SLICES):
            wd[i].wait()
            w_bf[pl.ds(i * ws, ws), :] = wbuf[i % 2].astype(jnp.bfloat16)
            if i + 2 < W_SLICES:
                wd[i + 2].start()

        s_val = sx_ref[0] * sw_ref[0]
        out_copies = [None, None]
        slot_ctr = [0]

        def gemm_store(chunk_f8, row_start):
            slot = slot_ctr[0] % 2
            slot_ctr[0] += 1
            acc = jnp.dot(chunk_f8.astype(jnp.bfloat16), w_bf[...],
                          preferred_element_type=jnp.float32)
            y = acc * s_val
            z = jnp.clip(y, -60.0, 60.0)
            if out_copies[slot] is not None:
                out_copies[slot].wait()
            stage_ref[slot] = (y / (1.0 + jnp.exp(-z))).astype(jnp.bfloat16)
            cp = pltpu.make_async_copy(
                stage_ref.at[slot],
                out_ref.at[pl.ds(row_start, half), :],
                out_sems.at[slot],
            )
            cp.start()
            out_copies[slot] = cp

        gemm_store(x_ref[pl.ds(0, half), :], my * m_per)
        gemm_store(x_ref[pl.ds(half, half), :], my * m_per + half)

        for h in range(N_HOP):
            for s in range(N_SEG):
                for d in range(2):
                    desc[d][h][s].wait_recv()
                    if h + 1 < N_HOP:
                        desc[d][h + 1][s].start()
            origin_r = lax.rem(my - (h + 1) + N_DEV, N_DEV)
            gemm_store(comm_r[h], origin_r * m_per)
            origin_l = lax.rem(my + (h + 1), N_DEV)
            gemm_store(comm_l[h], origin_l * m_per + half)

        for d in range(2):
            for h in range(N_HOP):
                for s in range(N_SEG):
                    desc[d][h][s].wait_send()
        for cp in out_copies:
            cp.wait()

    out_bf = pl.pallas_call(
        body,
        out_shape=jax.ShapeDtypeStruct((N_DEV * m_per, n_per), jnp.bfloat16),
        in_specs=[
            pl.BlockSpec(memory_space=pltpu.VMEM),
            pl.BlockSpec(memory_space=pl.ANY),
            pl.BlockSpec(memory_space=pltpu.SMEM),
            pl.BlockSpec(memory_space=pltpu.SMEM),
        ],
        out_specs=pl.BlockSpec(memory_space=pl.ANY),
        scratch_shapes=[
            pltpu.VMEM((k, n_per), jnp.bfloat16),
            pltpu.VMEM((2, ws, n_per), jnp.float32),
            pltpu.VMEM((N_HOP, half, k), jnp.float8_e5m2),
            pltpu.VMEM((N_HOP, half, k), jnp.float8_e5m2),
            pltpu.VMEM((2, half, n_per), jnp.bfloat16),
            pltpu.SemaphoreType.DMA((N_HOP, N_SEG)),
            pltpu.SemaphoreType.DMA((N_HOP, N_SEG)),
            pltpu.SemaphoreType.DMA((N_HOP, N_SEG)),
            pltpu.SemaphoreType.DMA((N_HOP, N_SEG)),
            pltpu.SemaphoreType.DMA((2,)),
            pltpu.SemaphoreType.DMA((2,)),
        ],
        compiler_params=pltpu.CompilerParams(collective_id=0),
    )(x8, w_mat, scale_x, scale_w)
    return out_bf.astype(jnp.float32)


# device time: 104026 ns/iter; 1.0044x vs baseline; 1.0044x over previous
import jax
import jax.numpy as jnp
from jax import lax
from jax.experimental import pallas as pl
from jax.experimental.pallas import tpu as pltpu

N_DEV = 8
N_HOP = N_DEV - 1
N_SEG = 2
W_SLICES = 8


def kernel(x, w_mat, scale_x, scale_w):
    m_per, k = x.shape
    k2, n_per = w_mat.shape
    assert k == k2
    half = m_per // 2
    seg = half // N_SEG
    ws = k // W_SLICES

    x8 = x.astype(jnp.float8_e5m2)

    def body(x_ref, w_hbm, sx_ref, sw_ref, out_ref,
             w_bf, wbuf, comm_r, comm_l, stage_ref,
             send_r, recv_r, send_l, recv_l, wsems, out_sems):
        my = lax.axis_index("i")
        left = lax.rem(my + N_DEV - 1, N_DEV)
        right = lax.rem(my + 1, N_DEV)

        barrier_sem = pltpu.get_barrier_semaphore()
        for nbr in (left, right):
            pl.semaphore_signal(
                barrier_sem, inc=1,
                device_id=(nbr,), device_id_type=pl.DeviceIdType.MESH,
            )
        pl.semaphore_wait(barrier_sem, 2)

        def mk(d, h, s):
            comm = comm_r if d == 0 else comm_l
            if h == 0:
                src = x_ref.at[pl.ds(d * half + s * seg, seg), :]
            else:
                src = comm.at[h - 1, pl.ds(s * seg, seg), :]
            return pltpu.make_async_remote_copy(
                src_ref=src,
                dst_ref=comm.at[h, pl.ds(s * seg, seg), :],
                send_sem=(send_r if d == 0 else send_l).at[h, s],
                recv_sem=(recv_r if d == 0 else recv_l).at[h, s],
                device_id=(right if d == 0 else left,),
                device_id_type=pl.DeviceIdType.MESH,
            )

        desc = [[[mk(d, h, s) for s in range(N_SEG)] for h in range(N_HOP)]
                for d in range(2)]
        for d in range(2):
            for s in range(N_SEG):
                desc[d][0][s].start()

        wd = [pltpu.make_async_copy(
                  w_hbm.at[pl.ds(i * ws, ws), :], wbuf.at[i % 2],
                  wsems.at[i % 2])
              for i in range(W_SLICES)]
        wd[0].start()
        wd[1].start()
        for i in range(W_SLICES):
            wd[i].wait()
            w_bf[pl.ds(i * ws, ws), :] = wbuf[i % 2].astype(jnp.bfloat16)
            if i + 2 < W_SLICES:
                wd[i + 2].start()

        s_val = sx_ref[0] * sw_ref[0]
        out_copies = [None, None]
        slot_ctr = [0]

        def gemm_store(chunk_f8, row_start):
            slot = slot_ctr[0] % 2
            slot_ctr[0] += 1
            acc = jnp.dot(chunk_f8.astype(jnp.bfloat16), w_bf[...],
                          preferred_element_type=jnp.float32)
            y = acc * s_val
            z = jnp.clip(y, -60.0, 60.0)
            if out_copies[slot] is not None:
                out_copies[slot].wait()
            stage_ref[slot] = (y / (1.0 + jnp.exp(-z))).astype(jnp.bfloat16)
            cp = pltpu.make_async_copy(
                stage_ref.at[slot],
                out_ref.at[pl.ds(row_start, half), :],
                out_sems.at[slot],
            )
            cp.start()
            out_copies[slot] = cp

        gemm_store(x_ref[pl.ds(0, half), :], my * m_per)
        gemm_store(x_ref[pl.ds(half, half), :], my * m_per + half)

        for h in range(N_HOP):
            for s in range(N_SEG):
                for d in range(2):
                    desc[d][h][s].wait_recv()
                    if h + 1 < N_HOP:
                        desc[d][h + 1][s].start()
            origin_r = lax.rem(my - (h + 1) + N_DEV, N_DEV)
            gemm_store(comm_r[h], origin_r * m_per)
            origin_l = lax.rem(my + (h + 1), N_DEV)
            gemm_store(comm_l[h], origin_l * m_per + half)

        for d in range(2):
            for h in range(N_HOP):
                for s in range(N_SEG):
                    desc[d][h][s].wait_send()
        for cp in out_copies:
            cp.wait()

    out_bf = pl.pallas_call(
        body,
        out_shape=jax.ShapeDtypeStruct((N_DEV * m_per, n_per), jnp.bfloat16),
        in_specs=[
            pl.BlockSpec(memory_space=pltpu.VMEM),
            pl.BlockSpec(memory_space=pl.ANY),
            pl.BlockSpec(memory_space=pltpu.SMEM),
            pl.BlockSpec(memory_space=pltpu.SMEM),
        ],
        out_specs=pl.BlockSpec(memory_space=pl.ANY),
        scratch_shapes=[
            pltpu.VMEM((k, n_per), jnp.bfloat16),
            pltpu.VMEM((2, ws, n_per), jnp.float32),
            pltpu.VMEM((N_HOP, half, k), jnp.float8_e5m2),
            pltpu.VMEM((N_HOP, half, k), jnp.float8_e5m2),
            pltpu.VMEM((2, half, n_per), jnp.bfloat16),
            pltpu.SemaphoreType.DMA((N_HOP, N_SEG)),
            pltpu.SemaphoreType.DMA((N_HOP, N_SEG)),
            pltpu.SemaphoreType.DMA((N_HOP, N_SEG)),
            pltpu.SemaphoreType.DMA((N_HOP, N_SEG)),
            pltpu.SemaphoreType.DMA((2,)),
            pltpu.SemaphoreType.DMA((2,)),
        ],
        compiler_params=pltpu.CompilerParams(collective_id=0),
    )(x8, w_mat, scale_x, scale_w)
    return out_bf.astype(jnp.float32)


# device time: 103216 ns/iter; 1.0123x vs baseline; 1.0078x over previous
import jax
import jax.numpy as jnp
from jax import lax
from jax.experimental import pallas as pl
from jax.experimental.pallas import tpu as pltpu

N_DEV = 8
N_HOP = N_DEV - 1
N_SEG = 2
W_SLICES = 8


def kernel(x, w_mat, scale_x, scale_w):
    m_per, k = x.shape
    k2, n_per = w_mat.shape
    assert k == k2
    half = m_per // 2
    seg = half // N_SEG
    ws = k // W_SLICES

    x8 = x.astype(jnp.float8_e5m2)

    def body(x_ref, w_hbm, sx_ref, sw_ref, out_ref,
             w_bf, wbuf, comm_r, comm_l, stage_ref,
             send_r, recv_r, send_l, recv_l, wsems, out_sems):
        my = lax.axis_index("i")
        left = lax.rem(my + N_DEV - 1, N_DEV)
        right = lax.rem(my + 1, N_DEV)

        barrier_sem = pltpu.get_barrier_semaphore()
        for nbr in (left, right):
            pl.semaphore_signal(
                barrier_sem, inc=1,
                device_id=(nbr,), device_id_type=pl.DeviceIdType.MESH,
            )
        pl.semaphore_wait(barrier_sem, 2)

        def mk(d, h, s):
            comm = comm_r if d == 0 else comm_l
            if h == 0:
                src = x_ref.at[pl.ds(d * half + s * seg, seg), :]
            else:
                src = comm.at[h - 1, pl.ds(s * seg, seg), :]
            return pltpu.make_async_remote_copy(
                src_ref=src,
                dst_ref=comm.at[h, pl.ds(s * seg, seg), :],
                send_sem=(send_r if d == 0 else send_l).at[h, s],
                recv_sem=(recv_r if d == 0 else recv_l).at[h, s],
                device_id=(right if d == 0 else left,),
                device_id_type=pl.DeviceIdType.MESH,
            )

        desc = [[[mk(d, h, s) for s in range(N_SEG)] for h in range(N_HOP)]
                for d in range(2)]
        for d in range(2):
            for s in range(N_SEG):
                desc[d][0][s].start()

        wd = [pltpu.make_async_copy(
                  w_hbm.at[pl.ds(i * ws, ws), :], wbuf.at[i % 2],
                  wsems.at[i % 2])
              for i in range(W_SLICES)]
        wd[0].start()
        wd[1].start()
        for i in range(W_SLICES):
            wd[i].wait()
            w_bf[pl.ds(i * ws, ws), :] = wbuf[i % 2].astype(jnp.bfloat16)
            if i + 2 < W_SLICES:
                wd[i + 2].start()

        s_val = sx_ref[0] * sw_ref[0]
        out_copies = [None, None]
        slot_ctr = [0]

        def gemm_store(chunk_f8, row_start):
            slot = slot_ctr[0] % 2
            slot_ctr[0] += 1
            acc = jnp.dot(chunk_f8.astype(jnp.bfloat16), w_bf[...],
                          preferred_element_type=jnp.float32)
            y = acc * s_val
            z = jnp.clip(y, -60.0, 60.0)
            if out_copies[slot] is not None:
                out_copies[slot].wait()
            stage_ref[slot] = y / (1.0 + jnp.exp(-z))
            cp = pltpu.make_async_copy(
                stage_ref.at[slot],
                out_ref.at[pl.ds(row_start, half), :],
                out_sems.at[slot],
            )
            cp.start()
            out_copies[slot] = cp

        gemm_store(x_ref[pl.ds(0, half), :], my * m_per)
        gemm_store(x_ref[pl.ds(half, half), :], my * m_per + half)

        for h in range(N_HOP):
            for s in range(N_SEG):
                for d in range(2):
                    desc[d][h][s].wait_recv()
                    if h + 1 < N_HOP:
                        desc[d][h + 1][s].start()
            origin_r = lax.rem(my - (h + 1) + N_DEV, N_DEV)
            gemm_store(comm_r[h], origin_r * m_per)
            origin_l = lax.rem(my + (h + 1), N_DEV)
            gemm_store(comm_l[h], origin_l * m_per + half)

        for d in range(2):
            for h in range(N_HOP):
                for s in range(N_SEG):
                    desc[d][h][s].wait_send()
        for cp in out_copies:
            cp.wait()

    return pl.pallas_call(
        body,
        out_shape=jax.ShapeDtypeStruct((N_DEV * m_per, n_per), jnp.float32),
        in_specs=[
            pl.BlockSpec(memory_space=pltpu.VMEM),
            pl.BlockSpec(memory_space=pl.ANY),
            pl.BlockSpec(memory_space=pltpu.SMEM),
            pl.BlockSpec(memory_space=pltpu.SMEM),
        ],
        out_specs=pl.BlockSpec(memory_space=pl.ANY),
        scratch_shapes=[
            pltpu.VMEM((k, n_per), jnp.bfloat16),
            pltpu.VMEM((2, ws, n_per), jnp.float32),
            pltpu.VMEM((N_HOP, half, k), jnp.float8_e5m2),
            pltpu.VMEM((N_HOP, half, k), jnp.float8_e5m2),
            pltpu.VMEM((2, half, n_per), jnp.float32),
            pltpu.SemaphoreType.DMA((N_HOP, N_SEG)),
            pltpu.SemaphoreType.DMA((N_HOP, N_SEG)),
            pltpu.SemaphoreType.DMA((N_HOP, N_SEG)),
            pltpu.SemaphoreType.DMA((N_HOP, N_SEG)),
            pltpu.SemaphoreType.DMA((2,)),
            pltpu.SemaphoreType.DMA((2,)),
        ],
        compiler_params=pltpu.CompilerParams(collective_id=0),
    )(x8, w_mat, scale_x, scale_w)
